# baseline (speedup 1.0000x reference)
import functools

import jax
import jax.numpy as jnp
from jax import lax
from jax.experimental import pallas as pl
from jax.experimental.pallas import tpu as pltpu
from jax.experimental.pallas import tpu_sc as plsc

NC = 2
NS = 16
NW = NC * NS
CH = 128
WDEG = 16
NBUF = 6
HID = 32
LANE = 16


def _chunk_base(w, full):
    return w * full + jnp.minimum(w, 4)


def _deg_body(full, rpt, dst_hbm, out_hbm, idx_v, ones_v, zb, deg_sh):
    c = lax.axis_index("c")
    s = lax.axis_index("s")
    w = c * NS + s
    base = _chunk_base(w, full)
    pltpu.sync_copy(dst_hbm.at[pl.ds(base, full)], idx_v.at[pl.ds(0, full)])

    @pl.when(w < 4)
    def _stage_extra():
        pltpu.sync_copy(dst_hbm.at[pl.ds(base + full, 1)],
                        idx_v.at[pl.ds(full, 1)])

    ones16 = jnp.ones((LANE,), jnp.float32)
    zeros16 = jnp.zeros((LANE,), jnp.float32)

    @pl.loop(0, CH)
    def _init_ones(i):
        ones_v[i, :] = ones16

    @pl.loop(0, rpt)
    def _init_zero(i):
        zb[i, :] = zeros16

    pltpu.sync_copy(zb, deg_sh.at[pl.ds(s * rpt, rpt)])
    plsc.subcore_barrier()

    @pl.loop(0, full)
    def _scatter(j):
        pltpu.sync_copy(ones_v, deg_sh.at[idx_v.at[j]], add=True)

    @pl.when(w < 4)
    def _scatter_extra():
        pltpu.sync_copy(ones_v, deg_sh.at[idx_v.at[full]], add=True)

    plsc.subcore_barrier()
    pltpu.sync_copy(deg_sh.at[pl.ds(s * rpt, rpt)], out_hbm.at[c, pl.ds(s * rpt, rpt)])


def _scat_body(full, rpt, src_hbm, dst_hbm, y_hbm, out_hbm,
               sidx, didx, bufs, zb, acc_sh, gsems, ssems):
    c = lax.axis_index("c")
    s = lax.axis_index("s")
    w = c * NS + s
    base = _chunk_base(w, full)
    pltpu.sync_copy(src_hbm.at[pl.ds(base, full)], sidx.at[pl.ds(0, full)])
    pltpu.sync_copy(dst_hbm.at[pl.ds(base, full)], didx.at[pl.ds(0, full)])

    @pl.when(w < 4)
    def _stage_extra():
        pltpu.sync_copy(src_hbm.at[pl.ds(base + full, 1)],
                        sidx.at[pl.ds(full, 1)])
        pltpu.sync_copy(dst_hbm.at[pl.ds(base + full, 1)],
                        didx.at[pl.ds(full, 1)])

    zeros16 = jnp.zeros((LANE,), jnp.float32)

    @pl.loop(0, rpt)
    def _init_zero(i):
        zb[i, 0:LANE] = zeros16
        zb[i, LANE:HID] = zeros16

    pltpu.sync_copy(zb, acc_sh.at[pl.ds(s * rpt, rpt)])
    plsc.subcore_barrier()

    def gather(j, b):
        pltpu.async_copy(y_hbm.at[sidx.at[j]], bufs.at[b], gsems.at[b])

    def gather_wait(j, b):
        pltpu.make_async_copy(y_hbm.at[sidx.at[j]], bufs.at[b], gsems.at[b]).wait()

    def scat(j, b):
        pltpu.async_copy(bufs.at[b], acc_sh.at[didx.at[j]], ssems.at[b], add=True)

    def scat_wait(j, b):
        pltpu.make_async_copy(bufs.at[b], acc_sh.at[didx.at[j]], ssems.at[b]).wait()

    for b in range(NBUF):
        gather(b, b)

    @pl.loop(0, full - NBUF, step=NBUF)
    def _group(j0):
        for b in range(NBUF):
            gather_wait(j0 + b, b)
            scat(j0 + b, b)
        for b in range(NBUF):
            scat_wait(j0 + b, b)
            gather(j0 + NBUF + b, b)

    j0 = full - NBUF
    for b in range(NBUF):
        gather_wait(j0 + b, b)
        scat(j0 + b, b)
    for b in range(NBUF):
        scat_wait(j0 + b, b)

    @pl.when(w < 4)
    def _tail():
        pltpu.sync_copy(y_hbm.at[sidx.at[full]], bufs.at[0])
        pltpu.sync_copy(bufs.at[0], acc_sh.at[didx.at[full]], add=True)

    plsc.subcore_barrier()
    pltpu.sync_copy(acc_sh.at[pl.ds(s * rpt, rpt)], out_hbm.at[c, pl.ds(s * rpt, rpt)])


def _prescale_body(n, x_ref, w_ref, degs_ref, y_ref, dinv_ref):
    xw = jnp.dot(x_ref[...], w_ref[...], preferred_element_type=jnp.float32)
    d = degs_ref[0, :n, 0:1] + degs_ref[1, :n, 0:1] + 1.0
    dinv = lax.rsqrt(d)
    y_ref[...] = xw * dinv
    dinv_ref[...] = dinv


def _head_body(n, p_ref, y_ref, dinv_ref, bgcn_ref, wih_ref, bih_ref, bhh_ref,
               wr_ref, br_ref, wc_ref, bc_ref, risk_ref, conf_ref):
    dinv = dinv_ref[...]
    ssum = p_ref[0, :n, :] + p_ref[1, :n, :] + y_ref[...]
    g = ssum * dinv + bgcn_ref[...]
    h = jnp.maximum(g, 0.0)
    gi = lax.dot_general(h, wih_ref[...], (((1,), (1,)), ((), ())),
                         preferred_element_type=jnp.float32) + bih_ref[...]
    bhh = bhh_ref[...]
    r = jax.nn.sigmoid(gi[:, 0:HID] + bhh[:, 0:HID])
    z = jax.nn.sigmoid(gi[:, HID:2 * HID] + bhh[:, HID:2 * HID])
    nn = jnp.tanh(gi[:, 2 * HID:] + r * bhh[:, 2 * HID:])
    h2 = (1.0 - z) * nn
    risk_ref[...] = jax.nn.sigmoid(
        jnp.sum(h2 * wr_ref[...], axis=1, keepdims=True) + br_ref[...])
    conf_ref[...] = jax.nn.sigmoid(
        jnp.sum(h2 * wc_ref[...], axis=1, keepdims=True) + bc_ref[...])


def kernel(x, edge_index, W_gcn, b_gcn, W_ih, W_hh, b_ih, b_hh,
           W_risk, b_risk, W_conf, b_conf):
    n, in_ch = x.shape
    e = edge_index.shape[1]
    del W_hh

    chunks = e // CH
    full = chunks // NW
    extra = chunks - full * NW
    assert extra == 4 and e % CH == 0
    npad = -(-n // (NS * 8)) * (NS * 8)
    rpt = npad // NS

    src2d = edge_index[0].reshape(chunks, CH)
    dst2d = edge_index[1].reshape(chunks, CH)

    mesh = plsc.VectorSubcoreMesh(core_axis_name="c", subcore_axis_name="s",
                                  num_cores=NC, num_subcores=NS)
    sc_params = pltpu.CompilerParams(use_tc_tiling_on_sc=False)

    deg_k = pl.kernel(
        functools.partial(_deg_body, full, rpt),
        out_type=jax.ShapeDtypeStruct((NC, npad, WDEG), jnp.float32),
        mesh=mesh,
        scratch_types=[
            pltpu.VMEM((full + 1, CH), jnp.int32),
            pltpu.VMEM((CH, WDEG), jnp.float32),
            pltpu.VMEM((rpt, WDEG), jnp.float32),
            pltpu.VMEM_SHARED((npad, WDEG), jnp.float32),
        ],
        compiler_params=sc_params,
    )
    degs = deg_k(dst2d)

    y, dinv = pl.pallas_call(
        functools.partial(_prescale_body, n),
        out_shape=(jax.ShapeDtypeStruct((n, HID), jnp.float32),
                   jax.ShapeDtypeStruct((n, 1), jnp.float32)),
    )(x, W_gcn, degs)

    scat_k = pl.kernel(
        functools.partial(_scat_body, full, rpt),
        out_type=jax.ShapeDtypeStruct((NC, npad, HID), jnp.float32),
        mesh=mesh,
        scratch_types=[
            pltpu.VMEM((full + 1, CH), jnp.int32),
            pltpu.VMEM((full + 1, CH), jnp.int32),
            pltpu.VMEM((NBUF, CH, HID), jnp.float32),
            pltpu.VMEM((rpt, HID), jnp.float32),
            pltpu.VMEM_SHARED((npad, HID), jnp.float32),
            pltpu.SemaphoreType.DMA((NBUF,)),
            pltpu.SemaphoreType.DMA((NBUF,)),
        ],
        compiler_params=sc_params,
    )
    parts = scat_k(src2d, dst2d, y)

    risk, conf = pl.pallas_call(
        functools.partial(_head_body, n),
        out_shape=(jax.ShapeDtypeStruct((n, 1), jnp.float32),
                   jax.ShapeDtypeStruct((n, 1), jnp.float32)),
    )(parts, y, dinv, b_gcn.reshape(1, HID), W_ih, b_ih.reshape(1, 3 * HID),
      b_hh.reshape(1, 3 * HID), W_risk, b_risk.reshape(1, 1),
      W_conf, b_conf.reshape(1, 1))
    return (risk, conf)


# speedup vs baseline: 1.0602x; 1.0602x over previous
import functools

import jax
import jax.numpy as jnp
from jax import lax
from jax.experimental import pallas as pl
from jax.experimental.pallas import tpu as pltpu
from jax.experimental.pallas import tpu_sc as plsc

NC = 2
NS = 16
NW = NC * NS
CH = 128
WDEG = 16
NBUF = 8
HID = 32


def _deg_body(chunks, rpt, ei_hbm, zeros_hbm, out_hbm, idx_v, ones_v, deg_sh):
    c = lax.axis_index("c")
    s = lax.axis_index("s")
    w = c * NS + s
    pltpu.sync_copy(ei_hbm.at[1, w], idx_v)
    ones16 = jnp.ones((WDEG,), jnp.float32)

    @pl.loop(0, CH)
    def _init_ones(i):
        ones_v[i, :] = ones16

    pltpu.sync_copy(zeros_hbm.at[pl.ds(s * rpt, rpt)], deg_sh.at[pl.ds(s * rpt, rpt)])
    plsc.subcore_barrier()

    @pl.loop(0, chunks)
    def _scatter(j):
        pltpu.sync_copy(ones_v, deg_sh.at[idx_v.at[j]], add=True)

    plsc.subcore_barrier()
    pltpu.sync_copy(deg_sh.at[pl.ds(s * rpt, rpt)], out_hbm.at[c, pl.ds(s * rpt, rpt)])


def _scat_body(chunks, rpt, ei_hbm, y_hbm, zeros_hbm, out_hbm,
               sidx, didx, bufs, acc_sh, gsems, ssems):
    c = lax.axis_index("c")
    s = lax.axis_index("s")
    w = c * NS + s
    pltpu.sync_copy(ei_hbm.at[0, w], sidx)
    pltpu.sync_copy(ei_hbm.at[1, w], didx)
    pltpu.sync_copy(zeros_hbm.at[pl.ds(s * rpt, rpt)], acc_sh.at[pl.ds(s * rpt, rpt)])
    plsc.subcore_barrier()

    def gather(j, b):
        pltpu.async_copy(y_hbm.at[sidx.at[j]], bufs.at[b], gsems.at[b])

    def gather_wait(j, b):
        pltpu.make_async_copy(y_hbm.at[sidx.at[j]], bufs.at[b], gsems.at[b]).wait()

    def scat(j, b):
        pltpu.async_copy(bufs.at[b], acc_sh.at[didx.at[j]], ssems.at[b], add=True)

    def scat_wait(j, b):
        pltpu.make_async_copy(bufs.at[b], acc_sh.at[didx.at[j]], ssems.at[b]).wait()

    for b in range(NBUF):
        gather(b, b)

    @pl.loop(0, chunks - NBUF, step=NBUF)
    def _group(j0):
        for b in range(NBUF):
            gather_wait(j0 + b, b)
            scat(j0 + b, b)
        for b in range(NBUF):
            scat_wait(j0 + b, b)
            gather(j0 + NBUF + b, b)

    j0 = chunks - NBUF
    for b in range(NBUF):
        gather_wait(j0 + b, b)
        scat(j0 + b, b)
    for b in range(NBUF):
        scat_wait(j0 + b, b)

    plsc.subcore_barrier()
    pltpu.sync_copy(acc_sh.at[pl.ds(s * rpt, rpt)], out_hbm.at[c, pl.ds(s * rpt, rpt)])


def _mm_body(x_ref, w_ref, xw_ref):
    xw_ref[...] = jnp.dot(x_ref[...], w_ref[...],
                          preferred_element_type=jnp.float32)


def _prescale_body(n, xw_ref, degs_ref, y_ref, dinv_ref):
    d = degs_ref[0, :n, 0:1] + degs_ref[1, :n, 0:1] + 1.0
    dinv = lax.rsqrt(d)
    y_ref[...] = xw_ref[...] * dinv
    dinv_ref[...] = dinv


def _head_body(n, p_ref, y_ref, dinv_ref, bgcn_ref, wih_ref, bih_ref, bhh_ref,
               wr_ref, br_ref, wc_ref, bc_ref, out_ref):
    dinv = dinv_ref[...]
    ssum = p_ref[0, :n, :] + p_ref[1, :n, :] + y_ref[...]
    g = ssum * dinv + bgcn_ref[...]
    h = jnp.maximum(g, 0.0)
    gi = lax.dot_general(h, wih_ref[...], (((1,), (1,)), ((), ())),
                         preferred_element_type=jnp.float32) + bih_ref[...]
    bhh = bhh_ref[...]
    r = jax.nn.sigmoid(gi[:, 0:HID] + bhh[:, 0:HID])
    z = jax.nn.sigmoid(gi[:, HID:2 * HID] + bhh[:, HID:2 * HID])
    nn = jnp.tanh(gi[:, 2 * HID:] + r * bhh[:, 2 * HID:])
    h2 = (1.0 - z) * nn
    out_ref[0, :, :] = jax.nn.sigmoid(
        jnp.sum(h2 * wr_ref[...], axis=1, keepdims=True) + br_ref[...])
    out_ref[1, :, :] = jax.nn.sigmoid(
        jnp.sum(h2 * wc_ref[...], axis=1, keepdims=True) + bc_ref[...])


def kernel(x, edge_index, W_gcn, b_gcn, W_ih, W_hh, b_ih, b_hh,
           W_risk, b_risk, W_conf, b_conf):
    n, in_ch = x.shape
    e = edge_index.shape[1]
    del W_hh

    chunks = -(-e // (NW * CH))
    chunks += (-chunks) % NBUF
    epad = NW * CH * chunks
    npad = -(-(n + 1) // (NS * 8)) * (NS * 8)
    rpt = npad // NS

    pad = epad - e
    ar = jnp.arange(pad, dtype=jnp.int32)
    junk = jnp.stack([ar % n, n + ar % (npad - n)])
    eip = jnp.concatenate([edge_index, junk], axis=1).reshape(2, NW, chunks, CH)

    zeros_deg = jnp.zeros((npad, WDEG), jnp.float32)
    zeros_acc = jnp.zeros((npad, HID), jnp.float32)

    mesh = plsc.VectorSubcoreMesh(core_axis_name="c", subcore_axis_name="s",
                                  num_cores=NC, num_subcores=NS)
    sc_params = pltpu.CompilerParams(use_tc_tiling_on_sc=False)

    deg_k = pl.kernel(
        functools.partial(_deg_body, chunks, rpt),
        out_type=jax.ShapeDtypeStruct((NC, npad, WDEG), jnp.float32),
        mesh=mesh,
        scratch_types=[
            pltpu.VMEM((chunks, CH), jnp.int32),
            pltpu.VMEM((CH, WDEG), jnp.float32),
            pltpu.VMEM_SHARED((npad, WDEG), jnp.float32),
        ],
        compiler_params=sc_params,
    )
    degs = deg_k(eip, zeros_deg)

    xw = pl.pallas_call(
        _mm_body,
        out_shape=jax.ShapeDtypeStruct((n, HID), jnp.float32),
    )(x, W_gcn)

    y, dinv = pl.pallas_call(
        functools.partial(_prescale_body, n),
        out_shape=(jax.ShapeDtypeStruct((n, HID), jnp.float32),
                   jax.ShapeDtypeStruct((n, 1), jnp.float32)),
    )(xw, degs)

    scat_k = pl.kernel(
        functools.partial(_scat_body, chunks, rpt),
        out_type=jax.ShapeDtypeStruct((NC, npad, HID), jnp.float32),
        mesh=mesh,
        scratch_types=[
            pltpu.VMEM((chunks, CH), jnp.int32),
            pltpu.VMEM((chunks, CH), jnp.int32),
            pltpu.VMEM((NBUF, CH, HID), jnp.float32),
            pltpu.VMEM_SHARED((npad, HID), jnp.float32),
            pltpu.SemaphoreType.DMA((NBUF,)),
            pltpu.SemaphoreType.DMA((NBUF,)),
        ],
        compiler_params=sc_params,
    )
    parts = scat_k(eip, y, zeros_acc)

    out = pl.pallas_call(
        functools.partial(_head_body, n),
        out_shape=jax.ShapeDtypeStruct((2, n, 1), jnp.float32),
    )(parts, y, dinv, b_gcn.reshape(1, HID), W_ih, b_ih.reshape(1, 3 * HID),
      b_hh.reshape(1, 3 * HID), W_risk, b_risk.reshape(1, 1),
      W_conf, b_conf.reshape(1, 1))
    return (out[0], out[1])
